# baseline (speedup 1.0000x reference)
import functools

import jax
import jax.numpy as jnp
from jax import lax
from jax.experimental import pallas as pl
from jax.experimental.pallas import tpu as pltpu
from jax.experimental.pallas import tpu_sc as plsc

B = 1024
N = 256
L = 16
NCHUNK = N // L
G = 4
CPG = NCHUNK // G
NC = 2
NS = 16
NW = NC * NS
BPW = B // NW
HPW = BPW // 2
BIG = 257.0 ** 2
INF = float("inf")


def _mesh():
    return plsc.VectorSubcoreMesh(
        core_axis_name="c", subcore_axis_name="s",
        num_cores=NC, num_subcores=NS)


@functools.partial(
    pl.kernel,
    out_type=jax.ShapeDtypeStruct((NW, L), jnp.float32),
    mesh=_mesh(),
    compiler_params=pltpu.CompilerParams(needs_layout_passes=False),
    scratch_types=[
        pltpu.VMEM((BPW, N), jnp.float32),
        pltpu.VMEM((BPW, N), jnp.float32),
        pltpu.VMEM((BPW, N), jnp.float32),
        pltpu.VMEM((BPW, N), jnp.float32),
        pltpu.VMEM((L,), jnp.float32),
    ],
)
def _greedy_match(ixt_hbm, iyt_hbm, tx_hbm, ty_hbm, out_hbm,
                  ixt_v, iyt_v, tx_v, ty_v, acc_v):
    wid = lax.axis_index("s") * NC + lax.axis_index("c")
    base = wid * BPW
    pltpu.sync_copy(ixt_hbm.at[pl.ds(base, BPW)], ixt_v)
    pltpu.sync_copy(iyt_hbm.at[pl.ds(base, BPW)], iyt_v)
    pltpu.sync_copy(tx_hbm.at[pl.ds(base, BPW)], tx_v)
    pltpu.sync_copy(ty_hbm.at[pl.ds(base, BPW)], ty_v)

    lanes = lax.iota(jnp.int32, L)
    zeros = jnp.zeros((L,), jnp.float32)
    infs = jnp.full((L,), INF, jnp.float32)
    perms = [lanes ^ s for s in (8, 4, 2, 1)]

    def allmin(v):
        for p in perms:
            v = jnp.minimum(v, v.at[p].get(mode="promise_in_bounds"))
        return v

    def stream_scan(row, jc, jl, exmask):
        txj = tx_v[row, pl.ds(jc, L)].at[jl].get(mode="promise_in_bounds")
        tyj = ty_v[row, pl.ds(jc, L)].at[jl].get(mode="promise_in_bounds")
        ms, cs = [], []
        for g in range(G):
            cm = infs
            cc = jnp.zeros((L,), jnp.int32)
            for c in range(g * CPG, (g + 1) * CPG):
                dx = txj - ixt_v[row, pl.ds(c * L, L)]
                dy = tyj - iyt_v[row, pl.ds(c * L, L)]
                d = dx * dx + dy * dy
                ok = jnp.left_shift(exmask, 31 - c) >= 0
                lt = (d < cm) & ok
                cc = jnp.where(lt, jnp.int32(c), cc)
                cm = jnp.where(lt, d, cm)
            ms.append(cm)
            cs.append(cc)
        lt1 = ms[1] < ms[0]
        m01 = jnp.where(lt1, ms[1], ms[0])
        c01 = jnp.where(lt1, cs[1], cs[0])
        lt2 = ms[3] < ms[2]
        m23 = jnp.where(lt2, ms[3], ms[2])
        c23 = jnp.where(lt2, cs[3], cs[2])
        lt3 = m23 < m01
        mf = jnp.where(lt3, m23, m01)
        cf = jnp.where(lt3, c23, c01)
        return mf, cf

    def stream_tail(mf, cf, exmask):
        m = allmin(mf)
        hit = m < BIG
        lffs = plsc.all_reduce_ffs(mf == m)
        lsel = jnp.where(hit, lffs, 0)
        csel = jnp.where(hit, cf, 0)
        bit = jnp.where(lanes == lsel,
                        jnp.left_shift(jnp.int32(1), csel), 0)
        return jnp.minimum(m, BIG), exmask | bit

    def batch_body(i, acc_vec):
        def step(j, carry):
            accb0, accb1, ex0, ex1 = carry
            jc = j & (N - L)
            jl = jnp.full((L,), j & (L - 1))
            mf0, cf0 = stream_scan(i, jc, jl, ex0)
            mf1, cf1 = stream_scan(i + HPW, jc, jl, ex1)
            se0, ex0 = stream_tail(mf0, cf0, ex0)
            se1, ex1 = stream_tail(mf1, cf1, ex1)
            return accb0 + se0, accb1 + se1, ex0, ex1

        izero = jnp.zeros((L,), jnp.int32)
        accb0, accb1, _, _ = lax.fori_loop(
            0, N, step, (zeros, zeros, izero, izero))
        return acc_vec + jnp.where(lanes == jnp.full((L,), i % L),
                                   accb0 + accb1, zeros)

    acc_vec = lax.fori_loop(0, HPW, batch_body, zeros)
    acc_v[...] = acc_vec
    pltpu.sync_copy(acc_v, out_hbm.at[wid])


def kernel(input, targets):
    inp = input.reshape(B, N, 2)
    tgt = targets.reshape(B, N, 2)
    ixt = inp[:, :, 0].reshape(B, L, NCHUNK).swapaxes(1, 2).reshape(B, N)
    iyt = inp[:, :, 1].reshape(B, L, NCHUNK).swapaxes(1, 2).reshape(B, N)
    partial = _greedy_match(ixt, iyt, tgt[:, :, 0], tgt[:, :, 1])
    return jnp.sum(partial) / B / 512.0


# speedup vs baseline: 1.1673x; 1.1673x over previous
import functools

import jax
import jax.numpy as jnp
from jax import lax
from jax.experimental import pallas as pl
from jax.experimental.pallas import tpu as pltpu
from jax.experimental.pallas import tpu_sc as plsc

B = 1024
N = 256
L = 16
NCHUNK = N // L
G = 4
CPG = NCHUNK // G
NC = 2
NS = 16
NW = NC * NS
BPW = B // NW
HPW = BPW // 2
BIG = 257.0 ** 2
INF = float("inf")


def _mesh():
    return plsc.VectorSubcoreMesh(
        core_axis_name="c", subcore_axis_name="s",
        num_cores=NC, num_subcores=NS)


@functools.partial(
    pl.kernel,
    out_type=jax.ShapeDtypeStruct((NW, L), jnp.float32),
    mesh=_mesh(),
    compiler_params=pltpu.CompilerParams(needs_layout_passes=False),
    scratch_types=[
        pltpu.VMEM((BPW, N), jnp.float32),
        pltpu.VMEM((BPW, N), jnp.float32),
        pltpu.VMEM((BPW, N), jnp.float32),
        pltpu.VMEM((BPW, N), jnp.float32),
        pltpu.VMEM((L,), jnp.float32),
    ],
)
def _greedy_match(ixt_hbm, iyt_hbm, tx_hbm, ty_hbm, out_hbm,
                  ixt_v, iyt_v, tx_v, ty_v, acc_v):
    wid = lax.axis_index("s") * NC + lax.axis_index("c")
    base = wid * BPW
    pltpu.sync_copy(ixt_hbm.at[pl.ds(base, BPW)], ixt_v)
    pltpu.sync_copy(iyt_hbm.at[pl.ds(base, BPW)], iyt_v)
    pltpu.sync_copy(tx_hbm.at[pl.ds(base, BPW)], tx_v)
    pltpu.sync_copy(ty_hbm.at[pl.ds(base, BPW)], ty_v)

    lanes = lax.iota(jnp.int32, L)
    zeros = jnp.zeros((L,), jnp.float32)
    infs = jnp.full((L,), INF, jnp.float32)
    perms = [lanes ^ s for s in (8, 4, 2, 1)]

    def allmin(v):
        for p in perms:
            v = jnp.minimum(v, v.at[p].get(mode="promise_in_bounds"))
        return v

    def stream_scan(row, jc, jl, exmask):
        txj = tx_v[row, pl.ds(jc, L)].at[jl].get(mode="promise_in_bounds")
        tyj = ty_v[row, pl.ds(jc, L)].at[jl].get(mode="promise_in_bounds")
        ms, cs = [], []
        for g in range(G):
            cm = infs
            cc = jnp.zeros((L,), jnp.int32)
            for c in range(g * CPG, (g + 1) * CPG):
                dx = txj - ixt_v[row, pl.ds(c * L, L)]
                dy = tyj - iyt_v[row, pl.ds(c * L, L)]
                d = dx * dx + dy * dy
                ok = jnp.left_shift(exmask, 31 - c) >= 0
                lt = (d < cm) & ok
                cc = jnp.where(lt, jnp.int32(c), cc)
                cm = jnp.where(lt, d, cm)
            ms.append(cm)
            cs.append(cc)
        lt1 = ms[1] < ms[0]
        m01 = jnp.where(lt1, ms[1], ms[0])
        c01 = jnp.where(lt1, cs[1], cs[0])
        lt2 = ms[3] < ms[2]
        m23 = jnp.where(lt2, ms[3], ms[2])
        c23 = jnp.where(lt2, cs[3], cs[2])
        lt3 = m23 < m01
        mf = jnp.where(lt3, m23, m01)
        cf = jnp.where(lt3, c23, c01)
        return mf, cf

    def stream_tail(mf, cf, exmask):
        m = allmin(mf)
        hit = m < BIG
        lffs = plsc.all_reduce_ffs(mf == m)
        lsel = jnp.where(hit, lffs, 0)
        csel = jnp.where(hit, cf, 0)
        bit = jnp.where(lanes == lsel,
                        jnp.left_shift(jnp.int32(1), csel), 0)
        return jnp.minimum(m, BIG), exmask | bit

    def batch_body(i, acc_vec):
        QPW = BPW // 4

        def step(j, carry):
            accs, exs = carry
            jc = j & (N - L)
            jl = jnp.full((L,), j & (L - 1))
            res = [stream_scan(i + s * QPW, jc, jl, exs[s])
                   for s in range(4)]
            new_accs, new_exs = [], []
            for s in range(4):
                se, ex = stream_tail(res[s][0], res[s][1], exs[s])
                new_accs.append(accs[s] + se)
                new_exs.append(ex)
            return tuple(new_accs), tuple(new_exs)

        izero = jnp.zeros((L,), jnp.int32)
        accs, _ = lax.fori_loop(
            0, N, step, ((zeros,) * 4, (izero,) * 4))
        return acc_vec + jnp.where(lanes == jnp.full((L,), i % L),
                                   accs[0] + accs[1] + accs[2] + accs[3],
                                   zeros)

    acc_vec = lax.fori_loop(0, BPW // 4, batch_body, zeros)
    acc_v[...] = acc_vec
    pltpu.sync_copy(acc_v, out_hbm.at[wid])


def kernel(input, targets):
    inp = input.reshape(B, N, 2)
    tgt = targets.reshape(B, N, 2)
    ixt = inp[:, :, 0].reshape(B, L, NCHUNK).swapaxes(1, 2).reshape(B, N)
    iyt = inp[:, :, 1].reshape(B, L, NCHUNK).swapaxes(1, 2).reshape(B, N)
    partial = _greedy_match(ixt, iyt, tgt[:, :, 0], tgt[:, :, 1])
    return jnp.sum(partial) / B / 512.0


# speedup vs baseline: 1.1673x; 1.0000x over previous
import functools

import jax
import jax.numpy as jnp
from jax import lax
from jax.experimental import pallas as pl
from jax.experimental.pallas import tpu as pltpu
from jax.experimental.pallas import tpu_sc as plsc

B = 1024
N = 256
L = 16
NCHUNK = N // L
G = 4
CPG = NCHUNK // G
NC = 2
NS = 16
NW = NC * NS
BPW = B // NW
NSTR = 4
QPW = BPW // NSTR
BIG = 257.0 ** 2
INF = float("inf")


def _mesh():
    return plsc.VectorSubcoreMesh(
        core_axis_name="c", subcore_axis_name="s",
        num_cores=NC, num_subcores=NS)


@functools.partial(
    pl.kernel,
    out_type=jax.ShapeDtypeStruct((NW, L), jnp.float32),
    mesh=_mesh(),
    compiler_params=pltpu.CompilerParams(needs_layout_passes=False),
    scratch_types=[
        pltpu.VMEM((BPW, N), jnp.float32),
        pltpu.VMEM((BPW, N), jnp.float32),
        pltpu.VMEM((BPW, N), jnp.float32),
        pltpu.VMEM((BPW, N), jnp.float32),
        pltpu.VMEM((L,), jnp.float32),
    ],
)
def _greedy_match(ixt_hbm, iyt_hbm, tx_hbm, ty_hbm, out_hbm,
                  ixt_v, iyt_v, tx_v, ty_v, acc_v):
    wid = lax.axis_index("s") * NC + lax.axis_index("c")
    base = wid * BPW
    pltpu.sync_copy(ixt_hbm.at[pl.ds(base, BPW)], ixt_v)
    pltpu.sync_copy(iyt_hbm.at[pl.ds(base, BPW)], iyt_v)
    pltpu.sync_copy(tx_hbm.at[pl.ds(base, BPW)], tx_v)
    pltpu.sync_copy(ty_hbm.at[pl.ds(base, BPW)], ty_v)

    lanes = lax.iota(jnp.int32, L)
    zeros = jnp.zeros((L,), jnp.float32)
    infs = jnp.full((L,), INF, jnp.float32)
    perms = [lanes ^ s for s in (8, 4, 2, 1)]

    def allmin(v):
        for p in perms:
            v = jnp.minimum(v, v.at[p].get(mode="promise_in_bounds"))
        return v

    def stream_scan(row, jc, jl, exmask):
        txj = tx_v[row, pl.ds(jc, L)].at[jl].get(mode="promise_in_bounds")
        tyj = ty_v[row, pl.ds(jc, L)].at[jl].get(mode="promise_in_bounds")
        ms, cs = [], []
        for g in range(G):
            cm = infs
            cc = jnp.zeros((L,), jnp.int32)
            for c in range(g * CPG, (g + 1) * CPG):
                dx = txj - ixt_v[row, pl.ds(c * L, L)]
                dy = tyj - iyt_v[row, pl.ds(c * L, L)]
                d = dx * dx + dy * dy
                ok = jnp.left_shift(exmask, 31 - c) >= 0
                lt = (d < cm) & ok
                cc = jnp.where(lt, jnp.int32(c), cc)
                cm = jnp.where(lt, d, cm)
            ms.append(cm)
            cs.append(cc)
        lt1 = ms[1] < ms[0]
        m01 = jnp.where(lt1, ms[1], ms[0])
        c01 = jnp.where(lt1, cs[1], cs[0])
        lt2 = ms[3] < ms[2]
        m23 = jnp.where(lt2, ms[3], ms[2])
        c23 = jnp.where(lt2, cs[3], cs[2])
        lt3 = m23 < m01
        mf = jnp.where(lt3, m23, m01)
        cf = jnp.where(lt3, c23, c01)
        return mf, cf

    def stream_tail(mf, cf, exmask):
        m = allmin(mf)
        hit = m < BIG
        lffs = plsc.all_reduce_ffs(mf == m)
        lsel = jnp.where(hit, lffs, 0)
        csel = jnp.where(hit, cf, 0)
        bit = jnp.where(lanes == lsel,
                        jnp.left_shift(jnp.int32(1), csel), 0)
        return jnp.minimum(m, BIG), exmask | bit

    def batch_body(i, acc_vec):
        def step(j, carry):
            accs, exs = carry
            jc = j & (N - L)
            jl = jnp.full((L,), j & (L - 1))
            res = [stream_scan(i + s * QPW, jc, jl, exs[s])
                   for s in range(NSTR)]
            new_accs, new_exs = [], []
            for s in range(NSTR):
                se, ex = stream_tail(res[s][0], res[s][1], exs[s])
                new_accs.append(accs[s] + se)
                new_exs.append(ex)
            return tuple(new_accs), tuple(new_exs)

        izero = jnp.zeros((L,), jnp.int32)
        accs, _ = lax.fori_loop(
            0, N, step, ((zeros,) * NSTR, (izero,) * NSTR))
        return acc_vec + jnp.where(lanes == jnp.full((L,), i % L),
                                   sum(accs[1:], accs[0]), zeros)

    acc_vec = lax.fori_loop(0, BPW // NSTR, batch_body, zeros)
    acc_v[...] = acc_vec
    pltpu.sync_copy(acc_v, out_hbm.at[wid])


def kernel(input, targets):
    inp = input.reshape(B, N, 2)
    tgt = targets.reshape(B, N, 2)
    ixt = inp[:, :, 0].reshape(B, L, NCHUNK).swapaxes(1, 2).reshape(B, N)
    iyt = inp[:, :, 1].reshape(B, L, NCHUNK).swapaxes(1, 2).reshape(B, N)
    partial = _greedy_match(ixt, iyt, tgt[:, :, 0], tgt[:, :, 1])
    return jnp.sum(partial) / B / 512.0


# speedup vs baseline: 1.1840x; 1.0143x over previous
import functools

import jax
import jax.numpy as jnp
from jax import lax
from jax.experimental import pallas as pl
from jax.experimental.pallas import tpu as pltpu
from jax.experimental.pallas import tpu_sc as plsc

B = 1024
N = 256
L = 16
NCHUNK = N // L
G = 4
CPG = NCHUNK // G
NC = 2
NS = 16
NW = NC * NS
BPW = B // NW
NSTR = 8
QPW = BPW // NSTR
BIG = 257.0 ** 2
INF = float("inf")


def _mesh():
    return plsc.VectorSubcoreMesh(
        core_axis_name="c", subcore_axis_name="s",
        num_cores=NC, num_subcores=NS)


@functools.partial(
    pl.kernel,
    out_type=jax.ShapeDtypeStruct((NW, L), jnp.float32),
    mesh=_mesh(),
    compiler_params=pltpu.CompilerParams(needs_layout_passes=False),
    scratch_types=[
        pltpu.VMEM((BPW, N), jnp.float32),
        pltpu.VMEM((BPW, N), jnp.float32),
        pltpu.VMEM((BPW, N), jnp.float32),
        pltpu.VMEM((BPW, N), jnp.float32),
        pltpu.VMEM((L,), jnp.float32),
    ],
)
def _greedy_match(ixt_hbm, iyt_hbm, tx_hbm, ty_hbm, out_hbm,
                  ixt_v, iyt_v, tx_v, ty_v, acc_v):
    wid = lax.axis_index("s") * NC + lax.axis_index("c")
    base = wid * BPW
    pltpu.sync_copy(ixt_hbm.at[pl.ds(base, BPW)], ixt_v)
    pltpu.sync_copy(iyt_hbm.at[pl.ds(base, BPW)], iyt_v)
    pltpu.sync_copy(tx_hbm.at[pl.ds(base, BPW)], tx_v)
    pltpu.sync_copy(ty_hbm.at[pl.ds(base, BPW)], ty_v)

    lanes = lax.iota(jnp.int32, L)
    zeros = jnp.zeros((L,), jnp.float32)
    infs = jnp.full((L,), INF, jnp.float32)
    perms = [lanes ^ s for s in (8, 4, 2, 1)]

    def allmin(v):
        for p in perms:
            v = jnp.minimum(v, v.at[p].get(mode="promise_in_bounds"))
        return v

    def stream_scan(row, jc, jl, exmask):
        txj = tx_v[row, pl.ds(jc, L)].at[jl].get(mode="promise_in_bounds")
        tyj = ty_v[row, pl.ds(jc, L)].at[jl].get(mode="promise_in_bounds")
        ms, cs = [], []
        for g in range(G):
            cm = infs
            cc = jnp.zeros((L,), jnp.int32)
            for c in range(g * CPG, (g + 1) * CPG):
                dx = txj - ixt_v[row, pl.ds(c * L, L)]
                dy = tyj - iyt_v[row, pl.ds(c * L, L)]
                d = dx * dx + dy * dy
                ok = jnp.left_shift(exmask, 31 - c) >= 0
                lt = (d < cm) & ok
                cc = jnp.where(lt, jnp.int32(c), cc)
                cm = jnp.where(lt, d, cm)
            ms.append(cm)
            cs.append(cc)
        lt1 = ms[1] < ms[0]
        m01 = jnp.where(lt1, ms[1], ms[0])
        c01 = jnp.where(lt1, cs[1], cs[0])
        lt2 = ms[3] < ms[2]
        m23 = jnp.where(lt2, ms[3], ms[2])
        c23 = jnp.where(lt2, cs[3], cs[2])
        lt3 = m23 < m01
        mf = jnp.where(lt3, m23, m01)
        cf = jnp.where(lt3, c23, c01)
        return mf, cf

    def stream_tail(mf, cf, exmask):
        m = allmin(mf)
        hit = m < BIG
        lffs = plsc.all_reduce_ffs(mf == m)
        lsel = jnp.where(hit, lffs, 0)
        csel = jnp.where(hit, cf, 0)
        bit = jnp.where(lanes == lsel,
                        jnp.left_shift(jnp.int32(1), csel), 0)
        return jnp.minimum(m, BIG), exmask | bit

    def batch_body(i, acc_vec):
        def step(j, carry):
            accs, exs = carry
            jc = j & (N - L)
            jl = jnp.full((L,), j & (L - 1))
            res = [stream_scan(i + s * QPW, jc, jl, exs[s])
                   for s in range(NSTR)]
            new_accs, new_exs = [], []
            for s in range(NSTR):
                se, ex = stream_tail(res[s][0], res[s][1], exs[s])
                new_accs.append(accs[s] + se)
                new_exs.append(ex)
            return tuple(new_accs), tuple(new_exs)

        izero = jnp.zeros((L,), jnp.int32)
        accs, _ = lax.fori_loop(
            0, N, step, ((zeros,) * NSTR, (izero,) * NSTR))
        return acc_vec + jnp.where(lanes == jnp.full((L,), i % L),
                                   sum(accs[1:], accs[0]), zeros)

    acc_vec = lax.fori_loop(0, BPW // NSTR, batch_body, zeros)
    acc_v[...] = acc_vec
    pltpu.sync_copy(acc_v, out_hbm.at[wid])


def kernel(input, targets):
    inp = input.reshape(B, N, 2)
    tgt = targets.reshape(B, N, 2)
    ixt = inp[:, :, 0].reshape(B, L, NCHUNK).swapaxes(1, 2).reshape(B, N)
    iyt = inp[:, :, 1].reshape(B, L, NCHUNK).swapaxes(1, 2).reshape(B, N)
    partial = _greedy_match(ixt, iyt, tgt[:, :, 0], tgt[:, :, 1])
    return jnp.sum(partial) / B / 512.0
